# baseline (speedup 1.0000x reference)
import jax
import jax.numpy as jnp
from jax import lax
from jax.experimental import pallas as pl
from jax.experimental.pallas import tpu as pltpu
from jax.experimental.pallas import tpu_sc as plsc

N_NODES = 10000
DEG = 32
WIDTH = 128
BATCH = 1024
LANES = 16

NW = 32
NP = 10240
ROWS_W = NP // NW
BPW = BATCH // NW
TROWS = 1032
OCHUNK = 64


def _seed1_body(feats_ref, wagg_ref, bagg_ref, wltop_ref, tbl_ref, u_ref):
    feats = feats_ref[...]
    mc = jnp.maximum(
        jnp.dot(feats, wagg_ref[...], preferred_element_type=jnp.float32)
        + bagg_ref[...], 0.0)
    base = jnp.maximum(bagg_ref[...], 0.0)
    pad = jnp.zeros((TROWS - BATCH - 1, WIDTH), jnp.float32)
    mcx = jnp.concatenate([mc, base, pad], axis=0)
    bits = lax.bitcast_convert_type(
        mcx.astype(jnp.bfloat16).astype(jnp.float32), jnp.int32)
    tbl_ref[...] = (lax.shift_right_logical(bits[:, :WIDTH // 2], 16)
                    | (bits[:, WIDTH // 2:] & jnp.int32(-65536)))
    u_ref[...] = jnp.dot(feats, wltop_ref[...],
                         preferred_element_type=jnp.float32)


def _seed1(feats, W_agg, b_agg2, Wl_top):
    return pl.pallas_call(
        _seed1_body,
        out_shape=(jax.ShapeDtypeStruct((TROWS, WIDTH // 2), jnp.int32),
                   jax.ShapeDtypeStruct((BATCH, WIDTH), jnp.float32)),
    )(feats, W_agg, b_agg2, Wl_top)


def _seed2_body(u_ref, ids_ref, nbd_ref, su_ref, nbsel_ref):
    su_ref[...] = jnp.zeros((NP, WIDTH), jnp.float32)

    def body(i, carry):
        idx = ids_ref[i]
        su_ref[pl.ds(idx, 1), :] = u_ref[pl.ds(i, 1), :]
        nbsel_ref[pl.ds(i, 1), :] = nbd_ref[pl.ds(idx, 1), :]
        return carry

    lax.fori_loop(0, BATCH, body, 0)


def _seed2(u, ids, nbd):
    return pl.pallas_call(
        _seed2_body,
        in_specs=[
            pl.BlockSpec(memory_space=pltpu.VMEM),
            pl.BlockSpec(memory_space=pltpu.SMEM),
            pl.BlockSpec(memory_space=pltpu.VMEM),
        ],
        out_shape=(jax.ShapeDtypeStruct((NP, WIDTH), jnp.float32),
                   jax.ShapeDtypeStruct((BATCH, DEG), jnp.int32)),
    )(u, ids, nbd)


def _aggmax_all_body(tbl_hbm, nbdf_hbm, ids_hbm, out_hbm,
                     tbl_v, pos_v, nbd_v, out_v, ids_v, cb0):
    wid = lax.axis_index("s") * 2 + lax.axis_index("c")
    base = wid * ROWS_W
    pltpu.sync_copy(tbl_hbm, tbl_v)
    pltpu.sync_copy(ids_hbm, ids_v)
    nvalid = N_NODES * DEG - base * DEG

    @pl.when(nvalid >= ROWS_W * DEG)
    def _():
        pltpu.sync_copy(nbdf_hbm.at[pl.ds(base * DEG, ROWS_W * DEG)], nbd_v)

    @pl.when(nvalid < ROWS_W * DEG)
    def _():
        pltpu.sync_copy(
            nbdf_hbm.at[pl.ds(base * DEG, (N_NODES - (NW - 1) * ROWS_W) * DEG)],
            nbd_v.at[pl.ds(0, (N_NODES - (NW - 1) * ROWS_W) * DEG)])

    col = lax.iota(jnp.int32, LANES)

    def pos_init(i, carry):
        pos_v[pl.ds(i * LANES, LANES)] = jnp.full((LANES,), BATCH, jnp.int32)
        return carry

    lax.fori_loop(0, NP // LANES, pos_init, 0)

    def pos_fill(k, carry):
        vals = lax.broadcast(k * LANES, (LANES,)) + col
        plsc.store_scatter(pos_v.at[...], [ids_v[pl.ds(k * LANES, LANES)]], vals)
        return carry

    lax.fori_loop(0, BATCH // LANES, pos_fill, 0)

    nc = WIDTH // (2 * LANES)
    base_accs = tuple(
        plsc.bitcast(tbl_v[pl.ds(BATCH * (WIDTH // 2) + c * LANES, LANES)],
                     jnp.bfloat16)
        for c in range(nc))

    def node(t, j2, carry):
        j = j2
        zero = jnp.zeros((LANES,), jnp.int32)
        nmax = jnp.full((LANES,), N_NODES - 1, jnp.int32)
        a0 = jnp.minimum(jnp.maximum(nbd_v[pl.ds(j * DEG, LANES)], zero), nmax)
        a1 = jnp.minimum(jnp.maximum(nbd_v[pl.ds(j * DEG + LANES, LANES)], zero), nmax)
        p0 = plsc.load_gather(pos_v.at[...], [a0])
        p1 = plsc.load_gather(pos_v.at[...], [a1])
        m0 = p0 < BATCH
        m1 = p1 < BATCH
        plsc.store_compressed(cb0.at[pl.ds(0, LANES)], p0, mask=m0)
        plsc.store_compressed(cb0.at[pl.ds(LANES, LANES)], p1, mask=m1)
        ntot = jnp.sum(m0.astype(jnp.int32) + m1.astype(jnp.int32))
        n0s = plsc.all_reduce_population_count(m0)
        accs = base_accs

        def body(k, accs):
            ks = lax.broadcast(k, (LANES,))
            ks = jnp.where(ks < n0s, ks, ks - n0s + LANES)
            row = plsc.load_gather(cb0.at[...], [ks])
            rb = row * (WIDTH // 2) + col
            return tuple(
                jnp.maximum(accs[c], plsc.bitcast(
                    plsc.load_gather(tbl_v.at[...], [rb + c * LANES]),
                    jnp.bfloat16))
                for c in range(nc))

        accs = plsc.parallel_loop(0, ntot, 1, unroll=2, carry=accs)(lambda k, a: body(k, a))
        for c in range(nc):
            out_v[pl.ds(j * (WIDTH // 2) + c * LANES, LANES)] = \
                plsc.bitcast(accs[c], jnp.int32)
        return carry

    lax.fori_loop(0, ROWS_W, lambda j, cc: node(0, j, cc), 0)
    pltpu.sync_copy(out_v, out_hbm.at[pl.ds(base * (WIDTH // 2),
                                            ROWS_W * (WIDTH // 2))])


def _aggmax_all(tbl, nbdf, ids):
    return pl.kernel(
        _aggmax_all_body,
        out_type=jax.ShapeDtypeStruct((NP * (WIDTH // 2),), jnp.int32),
        mesh=plsc.VectorSubcoreMesh(core_axis_name="c", subcore_axis_name="s"),
        compiler_params=pltpu.CompilerParams(needs_layout_passes=False),
        scratch_types=[
            pltpu.VMEM((TROWS * (WIDTH // 2),), jnp.int32),
            pltpu.VMEM((NP,), jnp.int32),
            pltpu.VMEM((ROWS_W * DEG,), jnp.int32),
            pltpu.VMEM((ROWS_W * (WIDTH // 2),), jnp.int32),
            pltpu.VMEM((BATCH,), jnp.int32),
            pltpu.VMEM((2 * LANES,), jnp.int32),
        ],
    )(tbl, nbdf, ids)


def _mid_body(su_ref, agg_ref, wlbe_ref, wlbo_ref, blin_ref, wagg_ref,
              bagg_ref, h1_ref, msg1_ref):
    w = agg_ref[...]
    lo = lax.bitcast_convert_type(w << 16, jnp.float32)
    hi = lax.bitcast_convert_type(w & jnp.int32(-65536), jnp.float32)
    h = jnp.maximum(
        su_ref[...]
        + jnp.dot(lo, wlbe_ref[...], preferred_element_type=jnp.float32)
        + jnp.dot(hi, wlbo_ref[...], preferred_element_type=jnp.float32)
        + blin_ref[...], 0.0)
    nrm = jnp.sqrt(jnp.sum(h * h, axis=1, keepdims=True))
    h1 = h / jnp.maximum(nrm, 1e-12)
    h1_ref[...] = h1
    msg1_ref[...] = jnp.maximum(
        jnp.dot(h1, wagg_ref[...], preferred_element_type=jnp.float32)
        + bagg_ref[...], 0.0).T


def _mid(su, aggp2, Wlb_e, Wlb_o, b_lin2, W_agg, b_agg2):
    blk = 512
    return pl.pallas_call(
        _mid_body,
        grid=(NP // blk,),
        in_specs=[
            pl.BlockSpec((blk, WIDTH), lambda i: (i, 0)),
            pl.BlockSpec((blk, WIDTH // 2), lambda i: (i, 0)),
            pl.BlockSpec((WIDTH // 2, WIDTH), lambda i: (0, 0)),
            pl.BlockSpec((WIDTH // 2, WIDTH), lambda i: (0, 0)),
            pl.BlockSpec((1, WIDTH), lambda i: (0, 0)),
            pl.BlockSpec((WIDTH, WIDTH), lambda i: (0, 0)),
            pl.BlockSpec((1, WIDTH), lambda i: (0, 0)),
        ],
        out_specs=(pl.BlockSpec((blk, WIDTH), lambda i: (i, 0)),
                   pl.BlockSpec((WIDTH, blk), lambda i: (0, i))),
        out_shape=(jax.ShapeDtypeStruct((NP, WIDTH), jnp.float32),
                   jax.ShapeDtypeStruct((WIDTH, NP), jnp.float32)),
    )(su, aggp2, Wlb_e, Wlb_o, b_lin2, W_agg, b_agg2)


CPW = WIDTH // NW


def _layer2_body(msg1t_hbm, h1_hbm, nbt_hbm, ids_hbm, sel_out, aggt_out,
                 cols_v, nbt_v, outt_v, ids_v, sel_v, sem):
    wid = lax.axis_index("s") * 2 + lax.axis_index("c")
    pltpu.sync_copy(msg1t_hbm.at[pl.ds(wid * CPW, CPW)], cols_v)
    pltpu.sync_copy(nbt_hbm, nbt_v)
    pltpu.sync_copy(ids_hbm.at[pl.ds(wid * BPW, BPW)], ids_v)
    pltpu.async_copy(h1_hbm.at[ids_v], sel_v, sem).wait()
    pltpu.sync_copy(sel_v, sel_out.at[pl.ds(wid * BPW, BPW)])

    def group(g, carry):
        nbr = nbt_v[pl.ds(g * LANES, LANES)]
        accs = [plsc.load_gather(cols_v.at[...],
                                 [jnp.full((LANES,), c, jnp.int32), nbr])
                for c in range(CPW)]
        for r in range(1, DEG):
            nbr = nbt_v[pl.ds(r * BATCH + g * LANES, LANES)]
            for c in range(CPW):
                accs[c] = jnp.maximum(accs[c], plsc.load_gather(
                    cols_v.at[...],
                    [jnp.full((LANES,), c, jnp.int32), nbr]))
        for c in range(CPW):
            outt_v[pl.ds(c * BATCH + g * LANES, LANES)] = accs[c]
        return carry

    lax.fori_loop(0, BATCH // LANES, group, 0)
    pltpu.sync_copy(outt_v, aggt_out.at[pl.ds(wid * CPW * BATCH, CPW * BATCH)])


def _layer2(msg1t, h1, nbt, ids):
    return pl.kernel(
        _layer2_body,
        out_type=(jax.ShapeDtypeStruct((BATCH, WIDTH), jnp.float32),
                  jax.ShapeDtypeStruct((WIDTH * BATCH,), jnp.float32)),
        mesh=plsc.VectorSubcoreMesh(core_axis_name="c", subcore_axis_name="s"),
        compiler_params=pltpu.CompilerParams(needs_layout_passes=False),
        scratch_types=[
            pltpu.VMEM((CPW, NP), jnp.float32),
            pltpu.VMEM((DEG * BATCH,), jnp.int32),
            pltpu.VMEM((CPW * BATCH,), jnp.float32),
            pltpu.VMEM((BPW,), jnp.int32),
            pltpu.VMEM((BPW, WIDTH), jnp.float32),
            pltpu.SemaphoreType.DMA,
        ],
    )(msg1t, h1, nbt, ids)


def _head_body(sel_ref, aggt_ref, wltop_ref, wlbot_ref, blin_ref, out_ref):
    h = jnp.maximum(
        jnp.dot(sel_ref[...], wltop_ref[...], preferred_element_type=jnp.float32)
        + jnp.dot(aggt_ref[...].T, wlbot_ref[...],
                  preferred_element_type=jnp.float32)
        + blin_ref[...], 0.0)
    nrm = jnp.sqrt(jnp.sum(h * h, axis=1, keepdims=True))
    out_ref[...] = h / jnp.maximum(nrm, 1e-12)


def _head(sel, agg1, Wl_top, Wl_bot, b_lin2):
    return pl.pallas_call(
        _head_body,
        out_shape=jax.ShapeDtypeStruct((BATCH, WIDTH), jnp.float32),
    )(sel, agg1, Wl_top, Wl_bot, b_lin2)


def kernel(nbd, x, W_agg, b_agg, W_lin, b_lin):
    ids = x[:, 0].astype(jnp.int32)
    feats = x[:, 1:]
    Wl_top = W_lin[:WIDTH]
    Wl_bot = W_lin[WIDTH:]
    b_agg2 = b_agg.reshape(1, WIDTH)
    b_lin2 = b_lin.reshape(1, WIDTH)
    nbdf = nbd.reshape(-1)

    tbl, u = _seed1(feats, W_agg, b_agg2, Wl_top)
    aggp = _aggmax_all(tbl.reshape(-1), nbdf, ids)
    su, nbsel = _seed2(u, ids, nbd)
    h1, msg1t = _mid(su, aggp.reshape(NP, WIDTH // 2), Wl_bot[:WIDTH // 2],
                     Wl_bot[WIDTH // 2:], b_lin2, W_agg, b_agg2)
    nbt = nbsel.T.reshape(-1)
    sel, agg1t = _layer2(msg1t, h1, nbt, ids)
    return _head(sel, agg1t.reshape(WIDTH, BATCH), Wl_top, Wl_bot, b_lin2)


# speedup vs baseline: 1.0218x; 1.0218x over previous
import jax
import jax.numpy as jnp
from jax import lax
from jax.experimental import pallas as pl
from jax.experimental.pallas import tpu as pltpu
from jax.experimental.pallas import tpu_sc as plsc

N_NODES = 10000
DEG = 32
WIDTH = 128
BATCH = 1024
LANES = 16

NW = 32
NP = 10240
ROWS_W = NP // NW
BPW = BATCH // NW
TROWS = 1032


def _seed1_body(feats_ref, wagg_ref, bagg_ref, wltop_ref, tbl_ref, u_ref):
    feats = feats_ref[...]
    mc = jnp.maximum(
        jnp.dot(feats, wagg_ref[...], preferred_element_type=jnp.float32)
        + bagg_ref[...], 0.0)
    base = jnp.maximum(bagg_ref[...], 0.0)
    pad = jnp.zeros((TROWS - BATCH - 1, WIDTH), jnp.float32)
    mcx = jnp.concatenate([mc, base, pad], axis=0)
    bits = lax.bitcast_convert_type(
        mcx.astype(jnp.bfloat16).astype(jnp.float32), jnp.int32)
    tbl_ref[...] = (lax.shift_right_logical(bits[:, :WIDTH // 2], 16)
                    | (bits[:, WIDTH // 2:] & jnp.int32(-65536)))
    u_ref[...] = jnp.dot(feats, wltop_ref[...],
                         preferred_element_type=jnp.float32)


def _seed1(feats, W_agg, b_agg2, Wl_top):
    return pl.pallas_call(
        _seed1_body,
        out_shape=(jax.ShapeDtypeStruct((TROWS, WIDTH // 2), jnp.int32),
                   jax.ShapeDtypeStruct((BATCH, WIDTH), jnp.float32)),
    )(feats, W_agg, b_agg2, Wl_top)


def _seed2_body(u_ref, ids_ref, nbd_ref, su_ref, nbsel_ref):
    su_ref[...] = jnp.zeros((NP, WIDTH), jnp.float32)

    def body(i, carry):
        idx = ids_ref[i]
        su_ref[pl.ds(idx, 1), :] = u_ref[pl.ds(i, 1), :]
        nbsel_ref[pl.ds(i, 1), :] = nbd_ref[pl.ds(idx, 1), :]
        return carry

    lax.fori_loop(0, BATCH, body, 0)


def _seed2(u, ids, nbd):
    return pl.pallas_call(
        _seed2_body,
        in_specs=[
            pl.BlockSpec(memory_space=pltpu.VMEM),
            pl.BlockSpec(memory_space=pltpu.SMEM),
            pl.BlockSpec(memory_space=pltpu.VMEM),
        ],
        out_shape=(jax.ShapeDtypeStruct((NP, WIDTH), jnp.float32),
                   jax.ShapeDtypeStruct((BATCH, DEG), jnp.int32)),
    )(u, ids, nbd)


def _aggmax_all_body(tbl_hbm, nbdf_hbm, ids_hbm, out_hbm,
                     tbl_v, pos_v, nbd_v, out_v, ids_v, cb0):
    wid = lax.axis_index("s") * 2 + lax.axis_index("c")
    base = wid * ROWS_W
    pltpu.sync_copy(tbl_hbm, tbl_v)
    pltpu.sync_copy(ids_hbm, ids_v)
    nvalid = N_NODES * DEG - base * DEG

    @pl.when(nvalid >= ROWS_W * DEG)
    def _():
        pltpu.sync_copy(nbdf_hbm.at[pl.ds(base * DEG, ROWS_W * DEG)], nbd_v)

    @pl.when(nvalid < ROWS_W * DEG)
    def _():
        pltpu.sync_copy(
            nbdf_hbm.at[pl.ds(base * DEG, (N_NODES - (NW - 1) * ROWS_W) * DEG)],
            nbd_v.at[pl.ds(0, (N_NODES - (NW - 1) * ROWS_W) * DEG)])

    col = lax.iota(jnp.int32, LANES)

    def pos_init(i, carry):
        pos_v[pl.ds(i * LANES, LANES)] = jnp.full((LANES,), BATCH, jnp.int32)
        return carry

    lax.fori_loop(0, NP // LANES, pos_init, 0)

    def pos_fill(k, carry):
        vals = lax.broadcast(k * LANES, (LANES,)) + col
        plsc.store_scatter(pos_v.at[...], [ids_v[pl.ds(k * LANES, LANES)]], vals)
        return carry

    lax.fori_loop(0, BATCH // LANES, pos_fill, 0)

    nc = WIDTH // (2 * LANES)
    base_accs = tuple(
        plsc.bitcast(tbl_v[pl.ds(BATCH * (WIDTH // 2) + c * LANES, LANES)],
                     jnp.bfloat16)
        for c in range(nc))

    def node(j, carry):
        zero = jnp.zeros((LANES,), jnp.int32)
        nmax = jnp.full((LANES,), N_NODES - 1, jnp.int32)
        a0 = jnp.minimum(jnp.maximum(nbd_v[pl.ds(j * DEG, LANES)], zero), nmax)
        a1 = jnp.minimum(jnp.maximum(nbd_v[pl.ds(j * DEG + LANES, LANES)], zero), nmax)
        p0 = plsc.load_gather(pos_v.at[...], [a0])
        p1 = plsc.load_gather(pos_v.at[...], [a1])
        m0 = p0 < BATCH
        m1 = p1 < BATCH
        plsc.store_compressed(cb0.at[pl.ds(0, LANES)], p0, mask=m0)
        plsc.store_compressed(cb0.at[pl.ds(LANES, LANES)], p1, mask=m1)
        ntot = jnp.sum(m0.astype(jnp.int32) + m1.astype(jnp.int32))
        n0s = plsc.all_reduce_population_count(m0)
        accs = base_accs

        def body(k, accs):
            ks = lax.broadcast(k, (LANES,))
            ks = jnp.where(ks < n0s, ks, ks - n0s + LANES)
            row = plsc.load_gather(cb0.at[...], [ks])
            rb = row * (WIDTH // 2) + col
            return tuple(
                jnp.maximum(accs[c], plsc.bitcast(
                    plsc.load_gather(tbl_v.at[...], [rb + c * LANES]),
                    jnp.bfloat16))
                for c in range(nc))

        accs = lax.fori_loop(0, ntot, body, accs)
        for c in range(nc):
            out_v[pl.ds(j * (WIDTH // 2) + c * LANES, LANES)] = \
                plsc.bitcast(accs[c], jnp.int32)
        return carry

    lax.fori_loop(0, ROWS_W, node, 0)
    pltpu.sync_copy(out_v, out_hbm.at[pl.ds(base * (WIDTH // 2),
                                            ROWS_W * (WIDTH // 2))])


def _aggmax_all(tbl, nbdf, ids):
    return pl.kernel(
        _aggmax_all_body,
        out_type=jax.ShapeDtypeStruct((NP * (WIDTH // 2),), jnp.int32),
        mesh=plsc.VectorSubcoreMesh(core_axis_name="c", subcore_axis_name="s"),
        compiler_params=pltpu.CompilerParams(needs_layout_passes=False),
        scratch_types=[
            pltpu.VMEM((TROWS * (WIDTH // 2),), jnp.int32),
            pltpu.VMEM((NP,), jnp.int32),
            pltpu.VMEM((ROWS_W * DEG,), jnp.int32),
            pltpu.VMEM((ROWS_W * (WIDTH // 2),), jnp.int32),
            pltpu.VMEM((BATCH,), jnp.int32),
            pltpu.VMEM((2 * LANES,), jnp.int32),
        ],
    )(tbl, nbdf, ids)


def _mid_body(su_ref, agg_ref, wlbe_ref, wlbo_ref, blin_ref, wagg_ref,
              bagg_ref, h1_ref, msg1_ref):
    w = agg_ref[...]
    lo = lax.bitcast_convert_type(w << 16, jnp.float32)
    hi = lax.bitcast_convert_type(w & jnp.int32(-65536), jnp.float32)
    h = jnp.maximum(
        su_ref[...]
        + jnp.dot(lo, wlbe_ref[...], preferred_element_type=jnp.float32)
        + jnp.dot(hi, wlbo_ref[...], preferred_element_type=jnp.float32)
        + blin_ref[...], 0.0)
    nrm = jnp.sqrt(jnp.sum(h * h, axis=1, keepdims=True))
    h1 = h / jnp.maximum(nrm, 1e-12)
    h1_ref[...] = h1
    msg1_ref[...] = jnp.maximum(
        jnp.dot(h1, wagg_ref[...], preferred_element_type=jnp.float32)
        + bagg_ref[...], 0.0).T


def _mid(su, aggp2, Wlb_e, Wlb_o, b_lin2, W_agg, b_agg2):
    blk = 512
    return pl.pallas_call(
        _mid_body,
        grid=(NP // blk,),
        in_specs=[
            pl.BlockSpec((blk, WIDTH), lambda i: (i, 0)),
            pl.BlockSpec((blk, WIDTH // 2), lambda i: (i, 0)),
            pl.BlockSpec((WIDTH // 2, WIDTH), lambda i: (0, 0)),
            pl.BlockSpec((WIDTH // 2, WIDTH), lambda i: (0, 0)),
            pl.BlockSpec((1, WIDTH), lambda i: (0, 0)),
            pl.BlockSpec((WIDTH, WIDTH), lambda i: (0, 0)),
            pl.BlockSpec((1, WIDTH), lambda i: (0, 0)),
        ],
        out_specs=(pl.BlockSpec((blk, WIDTH), lambda i: (i, 0)),
                   pl.BlockSpec((WIDTH, blk), lambda i: (0, i))),
        out_shape=(jax.ShapeDtypeStruct((NP, WIDTH), jnp.float32),
                   jax.ShapeDtypeStruct((WIDTH, NP), jnp.float32)),
    )(su, aggp2, Wlb_e, Wlb_o, b_lin2, W_agg, b_agg2)


CPW = WIDTH // NW


def _layer2_body(msg1t_hbm, h1_hbm, nbt_hbm, ids_hbm, sel_out, aggt_out,
                 cols_v, nbt_v, outt_v, ids_v, sel_v, sem):
    wid = lax.axis_index("s") * 2 + lax.axis_index("c")
    pltpu.sync_copy(msg1t_hbm.at[pl.ds(wid * CPW, CPW)], cols_v)
    pltpu.sync_copy(nbt_hbm, nbt_v)
    pltpu.sync_copy(ids_hbm.at[pl.ds(wid * BPW, BPW)], ids_v)
    pltpu.async_copy(h1_hbm.at[ids_v], sel_v, sem).wait()
    pltpu.sync_copy(sel_v, sel_out.at[pl.ds(wid * BPW, BPW)])

    def group(g, carry):
        nbr = nbt_v[pl.ds(g * LANES, LANES)]
        accs = [plsc.load_gather(cols_v.at[...],
                                 [jnp.full((LANES,), c, jnp.int32), nbr])
                for c in range(CPW)]
        for r in range(1, DEG):
            nbr = nbt_v[pl.ds(r * BATCH + g * LANES, LANES)]
            for c in range(CPW):
                accs[c] = jnp.maximum(accs[c], plsc.load_gather(
                    cols_v.at[...],
                    [jnp.full((LANES,), c, jnp.int32), nbr]))
        for c in range(CPW):
            outt_v[pl.ds(c * BATCH + g * LANES, LANES)] = accs[c]
        return carry

    lax.fori_loop(0, BATCH // LANES, group, 0)
    pltpu.sync_copy(outt_v, aggt_out.at[pl.ds(wid * CPW * BATCH, CPW * BATCH)])


def _layer2(msg1t, h1, nbt, ids):
    return pl.kernel(
        _layer2_body,
        out_type=(jax.ShapeDtypeStruct((BATCH, WIDTH), jnp.float32),
                  jax.ShapeDtypeStruct((WIDTH * BATCH,), jnp.float32)),
        mesh=plsc.VectorSubcoreMesh(core_axis_name="c", subcore_axis_name="s"),
        compiler_params=pltpu.CompilerParams(needs_layout_passes=False),
        scratch_types=[
            pltpu.VMEM((CPW, NP), jnp.float32),
            pltpu.VMEM((DEG * BATCH,), jnp.int32),
            pltpu.VMEM((CPW * BATCH,), jnp.float32),
            pltpu.VMEM((BPW,), jnp.int32),
            pltpu.VMEM((BPW, WIDTH), jnp.float32),
            pltpu.SemaphoreType.DMA,
        ],
    )(msg1t, h1, nbt, ids)


def _head_body(sel_ref, aggt_ref, wltop_ref, wlbot_ref, blin_ref, out_ref):
    h = jnp.maximum(
        jnp.dot(sel_ref[...], wltop_ref[...], preferred_element_type=jnp.float32)
        + jnp.dot(aggt_ref[...].T, wlbot_ref[...],
                  preferred_element_type=jnp.float32)
        + blin_ref[...], 0.0)
    nrm = jnp.sqrt(jnp.sum(h * h, axis=1, keepdims=True))
    out_ref[...] = h / jnp.maximum(nrm, 1e-12)


def _head(sel, agg1, Wl_top, Wl_bot, b_lin2):
    return pl.pallas_call(
        _head_body,
        out_shape=jax.ShapeDtypeStruct((BATCH, WIDTH), jnp.float32),
    )(sel, agg1, Wl_top, Wl_bot, b_lin2)


def kernel(nbd, x, W_agg, b_agg, W_lin, b_lin):
    ids = x[:, 0].astype(jnp.int32)
    feats = x[:, 1:]
    Wl_top = W_lin[:WIDTH]
    Wl_bot = W_lin[WIDTH:]
    b_agg2 = b_agg.reshape(1, WIDTH)
    b_lin2 = b_lin.reshape(1, WIDTH)
    nbdf = nbd.reshape(-1)

    tbl, u = _seed1(feats, W_agg, b_agg2, Wl_top)
    aggp = _aggmax_all(tbl.reshape(-1), nbdf, ids)
    su, nbsel = _seed2(u, ids, nbd)
    h1, msg1t = _mid(su, aggp.reshape(NP, WIDTH // 2), Wl_bot[:WIDTH // 2],
                     Wl_bot[WIDTH // 2:], b_lin2, W_agg, b_agg2)
    nbt = nbsel.T.reshape(-1)
    sel, agg1t = _layer2(msg1t, h1, nbt, ids)
    return _head(sel, agg1t.reshape(WIDTH, BATCH), Wl_top, Wl_bot, b_lin2)
